# Initial kernel scaffold; baseline (speedup 1.0000x reference)
import jax
import jax.numpy as jnp
from jax.experimental import pallas as pl


def kernel(node_id, weight):
    raise NotImplementedError("write your pallas kernel here")


# speedup vs baseline: 1.4393x; 1.4393x over previous
import functools

import jax
import jax.numpy as jnp
from jax import lax
from jax.experimental import pallas as pl
from jax.experimental.pallas import tpu as pltpu
from jax.experimental.pallas import tpu_sc as plsc

NUM_NODES = 100000
H_DIM = 128

NC = 2
NS = 16
NW = NC * NS

CHUNK = 128
N_CHUNKS = 25
B_PAD = NW * N_CHUNKS * CHUNK

FULL_CHUNKS = NUM_NODES // CHUNK
TAIL_ROWS = NUM_NODES - FULL_CHUNKS * CHUNK


def _build():
    mesh = plsc.VectorSubcoreMesh(core_axis_name="c", subcore_axis_name="s")

    @functools.partial(
        pl.kernel,
        out_type=jax.ShapeDtypeStruct((NUM_NODES, H_DIM), jnp.float32),
        mesh=mesh,
        scratch_types=[
            pltpu.VMEM((N_CHUNKS, CHUNK), jnp.int32),
            pltpu.VMEM((CHUNK, H_DIM), jnp.float32),
            pltpu.SemaphoreType.DMA,
        ],
    )
    def gather_kernel(table_hbm, idx_hbm, out_hbm, idx_v, rows_v, sem):
        wid = lax.axis_index("s") * NC + lax.axis_index("c")
        pltpu.sync_copy(idx_hbm.at[wid], idx_v)

        @pl.loop(0, N_CHUNKS)
        def _(j):
            c = wid * N_CHUNKS + j
            row0 = c * CHUNK

            @pl.when(c < FULL_CHUNKS + 1)
            def _():
                pltpu.async_copy(table_hbm.at[idx_v.at[j]], rows_v, sem).wait()

            @pl.when(c < FULL_CHUNKS)
            def _():
                pltpu.sync_copy(rows_v, out_hbm.at[pl.ds(row0, CHUNK)])

            @pl.when(c == FULL_CHUNKS)
            def _():
                pltpu.sync_copy(
                    rows_v.at[pl.ds(0, TAIL_ROWS)],
                    out_hbm.at[pl.ds(FULL_CHUNKS * CHUNK, TAIL_ROWS)],
                )

    return gather_kernel


_GATHER = _build()


@jax.jit
def kernel(node_id, weight):
    idx = jnp.squeeze(node_id, axis=1)
    idx_pad = jnp.zeros((B_PAD,), jnp.int32).at[:NUM_NODES].set(idx)
    return _GATHER(weight, idx_pad.reshape(NW, N_CHUNKS, CHUNK))


# speedup vs baseline: 1.7408x; 1.2095x over previous
import functools

import jax
import jax.numpy as jnp
from jax import lax
from jax.experimental import pallas as pl
from jax.experimental.pallas import tpu as pltpu
from jax.experimental.pallas import tpu_sc as plsc

NUM_NODES = 100000
H_DIM = 128

NC = 2
NS = 16
NW = NC * NS

CHUNK = 128
N_CHUNKS = 25
B_PAD = NW * N_CHUNKS * CHUNK

FULL_CHUNKS = NUM_NODES // CHUNK
TAIL_ROWS = NUM_NODES - FULL_CHUNKS * CHUNK

NBUF = 4
LEAD = 2


def _build():
    mesh = plsc.VectorSubcoreMesh(core_axis_name="c", subcore_axis_name="s")

    @functools.partial(
        pl.kernel,
        out_type=jax.ShapeDtypeStruct((NUM_NODES, H_DIM), jnp.float32),
        mesh=mesh,
        scratch_types=[
            pltpu.VMEM((N_CHUNKS, CHUNK), jnp.int32),
            pltpu.VMEM((NBUF, CHUNK, H_DIM), jnp.float32),
            pltpu.SemaphoreType.DMA((NBUF,)),
            pltpu.SemaphoreType.DMA((NBUF,)),
        ],
    )
    def gather_kernel(table_hbm, idx_hbm, out_hbm, idx_v, rows_v, gsem, osem):
        wid = lax.axis_index("s") * NC + lax.axis_index("c")
        c0 = wid * N_CHUNKS
        pltpu.sync_copy(idx_hbm.at[wid], idx_v)

        def fire_gather(j):
            @pl.when(c0 + j <= FULL_CHUNKS)
            def _():
                b = lax.rem(j, NBUF)
                pltpu.async_copy(
                    table_hbm.at[idx_v.at[j]], rows_v.at[b], gsem.at[b]
                )

        for j in range(LEAD):
            fire_gather(j)

        @pl.loop(0, N_CHUNKS)
        def _(j):
            b = lax.rem(j, NBUF)
            c = c0 + j

            @pl.when(j + LEAD < N_CHUNKS)
            def _():
                @pl.when(
                    jnp.logical_and(j - LEAD >= 0, c - LEAD < FULL_CHUNKS)
                )
                def _():
                    b2 = lax.rem(j + LEAD, NBUF)
                    pltpu.make_async_copy(
                        rows_v.at[b2],
                        out_hbm.at[pl.ds((c - LEAD) * CHUNK, CHUNK)],
                        osem.at[b2],
                    ).wait()

                fire_gather(j + LEAD)

            @pl.when(c <= FULL_CHUNKS)
            def _():
                pltpu.make_async_copy(
                    table_hbm.at[idx_v.at[j]], rows_v.at[b], gsem.at[b]
                ).wait()

            @pl.when(c < FULL_CHUNKS)
            def _():
                pltpu.async_copy(
                    rows_v.at[b], out_hbm.at[pl.ds(c * CHUNK, CHUNK)],
                    osem.at[b],
                )

            @pl.when(c == FULL_CHUNKS)
            def _():
                pltpu.sync_copy(
                    rows_v.at[b].at[pl.ds(0, TAIL_ROWS)],
                    out_hbm.at[pl.ds(FULL_CHUNKS * CHUNK, TAIL_ROWS)],
                )

        for j in range(N_CHUNKS - NBUF, N_CHUNKS):
            @pl.when(c0 + j < FULL_CHUNKS)
            def _(j=j):
                b = j % NBUF
                pltpu.make_async_copy(
                    rows_v.at[b],
                    out_hbm.at[pl.ds((c0 + j) * CHUNK, CHUNK)],
                    osem.at[b],
                ).wait()

    return gather_kernel


_GATHER = _build()


@jax.jit
def kernel(node_id, weight):
    idx = jnp.squeeze(node_id, axis=1)
    idx_pad = jnp.zeros((B_PAD,), jnp.int32).at[:NUM_NODES].set(idx)
    return _GATHER(weight, idx_pad.reshape(NW, N_CHUNKS, CHUNK))
